# baseline (speedup 1.0000x reference)
import functools

import jax
import jax.numpy as jnp
from jax import lax
from jax.experimental import pallas as pl
from jax.experimental.pallas import tpu as pltpu
from jax.experimental.pallas import tpu_sc as plsc

B = 16384
H = 50
D = 16
V = 1000000
NC = 2
NS = 16
NW = NC * NS

_mesh = plsc.VectorSubcoreMesh(core_axis_name="c", subcore_axis_name="s")

NT = V // 8
CH_OUT = 8
CH_IN = 64
NCH_W = 488
NCH_EXTRA = NT // CH_OUT - NW * NCH_W
DBUF = 4


@functools.partial(
    pl.kernel,
    out_type=jax.ShapeDtypeStruct((NT, 128), jnp.float32),
    mesh=_mesh,
    scratch_types=[
        pltpu.VMEM((DBUF, CH_IN, D), jnp.float32),
        pltpu.VMEM((DBUF, CH_OUT, 128), jnp.float32),
        [pltpu.SemaphoreType.DMA] * DBUF,
        [pltpu.SemaphoreType.DMA] * DBUF,
    ],
    compiler_params=pltpu.CompilerParams(use_tc_tiling_on_sc=True),
)
def _depad(tab_hbm, out_hbm, in_v, out_v, isems, osems):
    wid = lax.axis_index("s") * NC + lax.axis_index("c")
    g0 = wid * NCH_W

    def start_in(g, b):
        pltpu.async_copy(
            tab_hbm.at[pl.ds(g * CH_IN, CH_IN)], in_v.at[b], isems[b]
        )

    def wait_in(g, b):
        pltpu.make_async_copy(
            tab_hbm.at[pl.ds(g * CH_IN, CH_IN)], in_v.at[b], isems[b]
        ).wait()

    def start_out(g, b):
        pltpu.async_copy(
            out_v.at[b], out_hbm.at[pl.ds(g * CH_OUT, CH_OUT)], osems[b]
        )

    def wait_out(g, b):
        pltpu.make_async_copy(
            out_v.at[b], out_hbm.at[pl.ds(g * CH_OUT, CH_OUT)], osems[b]
        ).wait()

    def repack(b):
        for j in range(CH_OUT):
            for r in range(8):
                out_v[b, j, pl.ds(16 * r, 16)] = in_v[b, 8 * j + r, :]

    for b in range(DBUF):
        start_in(g0 + b, b)

    def body(i, carry):
        for b in range(DBUF):
            g = g0 + i * DBUF + b
            wait_in(g, b)

            @pl.when(i > 0)
            def _():
                wait_out(g - DBUF, b)

            repack(b)
            start_out(g, b)
            start_in(g + DBUF, b)
        return carry

    lax.fori_loop(0, NCH_W // DBUF - 1, body, 0)

    for b in range(DBUF):
        g = g0 + NCH_W - DBUF + b
        wait_in(g, b)
        wait_out(g - DBUF, b)
        repack(b)
        start_out(g, b)
    for b in range(DBUF):
        wait_out(g0 + NCH_W - DBUF + b, b)

    @pl.when(wid < NCH_EXTRA)
    def _():
        g = NW * NCH_W + wid
        start_in(g, 0)
        wait_in(g, 0)
        repack(0)
        start_out(g, 0)
        wait_out(g, 0)


BAGS_PER_W = B // NW
IDS_PER_STREAM = 800
BAGS_PER_STREAM = IDS_PER_STREAM // H
NSTREAM = BAGS_PER_W // BAGS_PER_STREAM
NBUF = 4


def _tree_sum(vals):
    while len(vals) > 1:
        nxt = [a + b for a, b in zip(vals[::2], vals[1::2])]
        if len(vals) % 2:
            nxt.append(vals[-1])
        vals = nxt
    return vals[0]


@functools.partial(
    pl.kernel,
    out_type=jax.ShapeDtypeStruct((B, D), jnp.float32),
    mesh=_mesh,
    scratch_types=[
        pltpu.VMEM((NSTREAM, IDS_PER_STREAM), jnp.int32),
        pltpu.VMEM((NBUF, IDS_PER_STREAM, D), jnp.float32),
        pltpu.VMEM((BAGS_PER_W, D), jnp.float32),
        [pltpu.SemaphoreType.DMA] * NBUF,
    ],
    compiler_params=pltpu.CompilerParams(use_tc_tiling_on_sc=False),
)
def _emb_bag(ids_hbm, table_hbm, out_hbm, idx_v, rows_v, out_v, sems):
    wid = lax.axis_index("s") * NC + lax.axis_index("c")
    pltpu.sync_copy(ids_hbm.at[pl.ds(wid * NSTREAM, NSTREAM)], idx_v)

    for b in range(NBUF):
        pltpu.async_copy(table_hbm.at[idx_v.at[b]], rows_v.at[b], sems[b])

    def step(s, b):
        pltpu.make_async_copy(
            table_hbm.at[idx_v.at[s]], rows_v.at[b], sems[b]
        ).wait()

        def red(k, carry):
            acc = _tree_sum([rows_v[b, k * H + h] for h in range(H)])
            out_v[BAGS_PER_STREAM * s + k] = acc
            return carry

        lax.fori_loop(0, BAGS_PER_STREAM, red, 0)

    def chunk(i, carry):
        s0 = i * NBUF
        for b in range(NBUF):
            s = s0 + b
            step(s, b)
            pltpu.async_copy(
                table_hbm.at[idx_v.at[s + NBUF]], rows_v.at[b], sems[b]
            )
        return carry

    lax.fori_loop(0, NSTREAM // NBUF - 1, chunk, 0)
    for b in range(NBUF):
        step(NSTREAM - NBUF + b, b)

    pltpu.sync_copy(out_v, out_hbm.at[pl.ds(wid * BAGS_PER_W, BAGS_PER_W)])


def kernel(ids, table):
    ids2 = jnp.minimum(ids, V - 1).reshape(B * H // IDS_PER_STREAM, IDS_PER_STREAM)
    table_lin = _depad(table).reshape(V, D)
    return _emb_bag(ids2, table_lin)


# speedup vs baseline: 1.0773x; 1.0773x over previous
import functools

import jax
import jax.numpy as jnp
from jax import lax
from jax.experimental import pallas as pl
from jax.experimental.pallas import tpu as pltpu
from jax.experimental.pallas import tpu_sc as plsc

B = 16384
H = 50
D = 16
V = 1000000
NC = 2
NS = 16
NW = NC * NS
BAGS_PER_W = B // NW
IDS_PER_STREAM = 800
BAGS_PER_STREAM = IDS_PER_STREAM // H
NSTREAM = BAGS_PER_W // BAGS_PER_STREAM
NBUF = 4

_mesh = plsc.VectorSubcoreMesh(core_axis_name="c", subcore_axis_name="s")


def _tree_sum(vals):
    while len(vals) > 1:
        nxt = [a + b for a, b in zip(vals[::2], vals[1::2])]
        if len(vals) % 2:
            nxt.append(vals[-1])
        vals = nxt
    return vals[0]


@functools.partial(
    pl.kernel,
    out_type=jax.ShapeDtypeStruct((B, D), jnp.float32),
    mesh=_mesh,
    scratch_types=[
        pltpu.VMEM((NSTREAM, IDS_PER_STREAM), jnp.int32),
        pltpu.VMEM((NBUF, IDS_PER_STREAM, D), jnp.float32),
        pltpu.VMEM((BAGS_PER_W, D), jnp.float32),
        [pltpu.SemaphoreType.DMA] * NBUF,
    ],
    compiler_params=pltpu.CompilerParams(use_tc_tiling_on_sc=False),
)
def _emb_bag(ids_hbm, table_hbm, out_hbm, idx_v, rows_v, out_v, sems):
    wid = lax.axis_index("s") * NC + lax.axis_index("c")
    pltpu.sync_copy(ids_hbm.at[pl.ds(wid * NSTREAM, NSTREAM)], idx_v)

    for b in range(NBUF):
        pltpu.async_copy(table_hbm.at[idx_v.at[b]], rows_v.at[b], sems[b])

    def step(s, b):
        pltpu.make_async_copy(
            table_hbm.at[idx_v.at[s]], rows_v.at[b], sems[b]
        ).wait()

        def red(k, carry):
            acc = _tree_sum([rows_v[b, k * H + h] for h in range(H)])
            out_v[BAGS_PER_STREAM * s + k] = acc
            return carry

        lax.fori_loop(0, BAGS_PER_STREAM, red, 0)

    def chunk(i, carry):
        s0 = i * NBUF
        for b in range(NBUF):
            s = s0 + b
            step(s, b)
            pltpu.async_copy(
                table_hbm.at[idx_v.at[s + NBUF]], rows_v.at[b], sems[b]
            )
        return carry

    lax.fori_loop(0, NSTREAM // NBUF - 1, chunk, 0)
    for b in range(NBUF):
        step(NSTREAM - NBUF + b, b)

    pltpu.sync_copy(out_v, out_hbm.at[pl.ds(wid * BAGS_PER_W, BAGS_PER_W)])


def kernel(ids, table):
    ids2 = jnp.minimum(ids, V - 1).reshape(B * H // IDS_PER_STREAM, IDS_PER_STREAM)
    table_lin = table + jnp.float32(0.0)
    return _emb_bag(ids2, table_lin)
